# baseline (speedup 1.0000x reference)
import jax
import jax.numpy as jnp
import numpy as np
from jax import lax
from jax.experimental import pallas as pl
from jax.experimental.pallas import tpu as pltpu
from jax.experimental.pallas import tpu_sc as plsc

NC = 2
NS = 16
L = 16
NW = NC * NS

E = 320000
D = 128
EPW = E // NW
C = 80
NCHUNK = EPW // C


def _decoder_body(z_hbm, src_hbm, dst_hbm, out_hbm,
                  sidx, didx, outv,
                  srows0, srows1, drows0, drows1,
                  sg0, sg1, sd0, sd1):
    wid = lax.axis_index("s") * NC + lax.axis_index("c")
    base0 = wid * EPW
    lanes = lax.iota(jnp.int32, L)
    srows = (srows0, srows1)
    drows = (drows0, drows1)
    sg = (sg0, sg1)
    sd = (sd0, sd1)

    pltpu.sync_copy(src_hbm.at[pl.ds(base0, EPW)], sidx)
    pltpu.sync_copy(dst_hbm.at[pl.ds(base0, EPW)], didx)

    @pl.loop(0, EPW // L)
    def _zero(i):
        outv[pl.ds(i * L, L)] = jnp.zeros((L,), jnp.float32)

    def issue(c, p):
        pltpu.async_copy(z_hbm.at[sidx.at[pl.ds(c * C, C)]], srows[p], sg[p])
        pltpu.async_copy(z_hbm.at[didx.at[pl.ds(c * C, C)]], drows[p], sd[p])

    def wait(p):
        pltpu.make_async_copy(
            z_hbm.at[sidx.at[pl.ds(0, C)]], srows[p], sg[p]).wait()
        pltpu.make_async_copy(
            z_hbm.at[didx.at[pl.ds(0, C)]], drows[p], sd[p]).wait()

    def compute(c, p):
        sr, dr = srows[p], drows[p]

        for b in range(C // L):
            out_base = c * C + b * L
            for e in range(L):
                row = b * L + e
                pr = [sr[row, pl.ds(j * L, L)] * dr[row, pl.ds(j * L, L)]
                      for j in range(D // L)]
                while len(pr) > 1:
                    pr = [pr[k] + pr[k + 1] for k in range(0, len(pr) - 1, 2)] \
                        + ([pr[-1]] if len(pr) % 2 else [])
                plsc.addupdate_scatter(
                    outv, [jnp.full((L,), out_base + e, jnp.int32)], pr[0])
            acc = outv[pl.ds(out_base, L)]
            outv[pl.ds(out_base, L)] = 1.0 / (1.0 + jnp.exp(-acc))

    issue(0, 0)

    @pl.loop(0, (NCHUNK - 1) // 2)
    def _pair(t):
        c0 = 2 * t
        wait(0)
        issue(c0 + 1, 1)
        compute(c0, 0)
        wait(1)
        issue(c0 + 2, 0)
        compute(c0 + 1, 1)

    wait(0)
    compute(NCHUNK - 1, 0)
    pltpu.sync_copy(outv, out_hbm.at[pl.ds(base0, EPW)])


@jax.jit
def _run(z, src, dst):
    mesh = plsc.VectorSubcoreMesh(
        core_axis_name="c", subcore_axis_name="s",
        num_cores=NC, num_subcores=NS)
    f = pl.kernel(
        _decoder_body,
        out_type=jax.ShapeDtypeStruct((E,), jnp.float32),
        mesh=mesh,
        scratch_types=[
            pltpu.VMEM((EPW,), jnp.int32),
            pltpu.VMEM((EPW,), jnp.int32),
            pltpu.VMEM((EPW,), jnp.float32),
            pltpu.VMEM((C, D), jnp.float32),
            pltpu.VMEM((C, D), jnp.float32),
            pltpu.VMEM((C, D), jnp.float32),
            pltpu.VMEM((C, D), jnp.float32),
            pltpu.SemaphoreType.DMA,
            pltpu.SemaphoreType.DMA,
            pltpu.SemaphoreType.DMA,
            pltpu.SemaphoreType.DMA,
        ],
        compiler_params=pltpu.CompilerParams(needs_layout_passes=False),
    )
    return f(z, src, dst)


def kernel(z, edge_index):
    ei = edge_index.astype(jnp.int32)
    return _run(z, ei[0], ei[1])


# speedup vs baseline: 1.5279x; 1.5279x over previous
import jax
import jax.numpy as jnp
from jax import lax
from jax.experimental import pallas as pl
from jax.experimental.pallas import tpu as pltpu
from jax.experimental.pallas import tpu_sc as plsc

NC = 2
NS = 16
L = 16
NW = NC * NS

E = 320000
V = 10000
D = 128
EPW = E // NW
C = 80
NCHUNK = EPW // C
NPS = 640


def _tree_sum(vs):
    while len(vs) > 1:
        vs = [vs[k] + vs[k + 1] for k in range(0, len(vs) - 1, 2)] \
            + ([vs[-1]] if len(vs) % 2 else [])
    return vs[0]


def _decoder_body(z_hbm, src_hbm, dst_hbm, out_hbm,
                  sidx, didx, outv, norms,
                  srows0, srows1, srows2, shnorm, sg0, sg1, sg2):
    sid = lax.axis_index("s")
    wid = sid * NC + lax.axis_index("c")
    base0 = wid * EPW
    srows = (srows0, srows1, srows2)
    sg = (sg0, sg1, sg2)

    pltpu.sync_copy(src_hbm.at[pl.ds(base0, EPW)], sidx)
    pltpu.sync_copy(dst_hbm.at[pl.ds(base0, EPW)], didx)

    node0 = sid * NPS
    nmine = jnp.minimum(NPS, V - node0)
    zero = jnp.zeros((L,), jnp.float32)

    @pl.loop(0, nmine // L)
    def _nzero(i):
        norms[pl.ds(node0 + i * L, L)] = zero

    @pl.loop(0, nmine // C)
    def _nchunk(t):
        row0 = node0 + t * C
        pltpu.sync_copy(z_hbm.at[pl.ds(row0, C), :], srows0)

        @pl.loop(0, C)
        def _nrow(e):
            pr = [srows0[e, pl.ds(j * L, L)] for j in range(D // L)]
            pr = _tree_sum([v * v for v in pr])
            plsc.addupdate_scatter(
                norms, [jnp.full((L,), row0 + e, jnp.int32)], pr)

    pltpu.sync_copy(norms.at[pl.ds(node0, nmine)],
                    shnorm.at[pl.ds(node0, nmine)])
    plsc.subcore_barrier()
    pltpu.sync_copy(shnorm, norms)

    def issue1(c, p):
        pltpu.async_copy(z_hbm.at[sidx.at[pl.ds(c * C, C)]], srows[p], sg[p])

    def issue2(c, p):
        pltpu.async_copy(z_hbm.at[didx.at[pl.ds(c * C, C)]], srows[p], sg[p],
                         add=True)

    def wait(p):
        pltpu.make_async_copy(
            z_hbm.at[sidx.at[pl.ds(0, C)]], srows[p], sg[p]).wait()

    def compute(c, p):
        sr = srows[p]

        @pl.loop(0, C // L)
        def _blk(b):
            out_base = c * C + b * L
            for e in range(L):
                row = b * L + e
                pr = [sr[row, pl.ds(j * L, L)] for j in range(D // L)]
                pr = _tree_sum([v * v for v in pr])
                plsc.addupdate_scatter(
                    outv, [jnp.full((L,), out_base + e, jnp.int32)], pr)
            ns = plsc.load_gather(norms, [sidx[pl.ds(out_base, L)]])
            nd = plsc.load_gather(norms, [didx[pl.ds(out_base, L)]])
            val = 0.5 * (outv[pl.ds(out_base, L)] - ns - nd)
            outv[pl.ds(out_base, L)] = 1.0 / (1.0 + jnp.exp(-val))

    @pl.loop(0, EPW // L)
    def _zero(i):
        outv[pl.ds(i * L, L)] = zero

    issue1(0, 0)
    wait(0)
    issue2(0, 0)
    issue1(1, 1)

    def step(c, p):
        q = (p + 1) % 3
        r = (p + 2) % 3
        wait(q)
        issue2(c + 1, q)
        issue1(c + 2, r)
        wait(p)
        compute(c, p)

    @pl.loop(0, (NCHUNK - 2) // 3)
    def _trip(t):
        c0 = 3 * t
        step(c0, 0)
        step(c0 + 1, 1)
        step(c0 + 2, 2)

    ct = NCHUNK - 2
    pt = ct % 3
    qt = (pt + 1) % 3
    wait(qt)
    issue2(ct + 1, qt)
    wait(pt)
    compute(ct, pt)
    wait(qt)
    compute(ct + 1, qt)

    pltpu.sync_copy(outv, out_hbm.at[pl.ds(base0, EPW)])


@jax.jit
def _run(z, src, dst):
    mesh = plsc.VectorSubcoreMesh(
        core_axis_name="c", subcore_axis_name="s",
        num_cores=NC, num_subcores=NS)
    f = pl.kernel(
        _decoder_body,
        out_type=jax.ShapeDtypeStruct((E,), jnp.float32),
        mesh=mesh,
        scratch_types=[
            pltpu.VMEM((EPW,), jnp.int32),
            pltpu.VMEM((EPW,), jnp.int32),
            pltpu.VMEM((EPW,), jnp.float32),
            pltpu.VMEM((V,), jnp.float32),
            pltpu.VMEM((C, D), jnp.float32),
            pltpu.VMEM((C, D), jnp.float32),
            pltpu.VMEM((C, D), jnp.float32),
            pltpu.VMEM_SHARED((V,), jnp.float32),
            pltpu.SemaphoreType.DMA,
            pltpu.SemaphoreType.DMA,
            pltpu.SemaphoreType.DMA,
        ],
        compiler_params=pltpu.CompilerParams(needs_layout_passes=False),
    )
    return f(z, src, dst)


def kernel(z, edge_index):
    ei = edge_index.astype(jnp.int32)
    return _run(z, ei[0], ei[1])


# speedup vs baseline: 2.2727x; 1.4875x over previous
import jax
import jax.numpy as jnp
from jax import lax
from jax.experimental import pallas as pl
from jax.experimental.pallas import tpu as pltpu
from jax.experimental.pallas import tpu_sc as plsc

NC = 2
NS = 16
L = 16
NW = NC * NS

E = 320000
V = 10000
D = 128
EPW = E // NW
C = 80
NCHUNK = EPW // C
NPS = 640


def _tree_sum(vs):
    while len(vs) > 1:
        vs = [vs[k] + vs[k + 1] for k in range(0, len(vs) - 1, 2)] \
            + ([vs[-1]] if len(vs) % 2 else [])
    return vs[0]


def _edge_sq(sr, row):
    pr = [sr[row, pl.ds(j * L, L)] for j in range(D // L)]
    return _tree_sum([v * v for v in pr])


def _decoder_body(z_hbm, src_hbm, dst_hbm, out_hbm,
                  sidx, didx, outv, norms,
                  srows0, srows1, srows2, shnorm, sg0, sg1, sg2):
    sid = lax.axis_index("s")
    wid = sid * NC + lax.axis_index("c")
    base0 = wid * EPW
    lanes = lax.iota(jnp.int32, L)
    srows = (srows0, srows1, srows2)
    sg = (sg0, sg1, sg2)

    pltpu.sync_copy(src_hbm.at[pl.ds(base0, EPW)], sidx)
    pltpu.sync_copy(dst_hbm.at[pl.ds(base0, EPW)], didx)

    node0 = sid * NPS
    nmine = jnp.minimum(NPS, V - node0)
    zero = jnp.zeros((L,), jnp.float32)

    @pl.loop(0, nmine // L)
    def _nzero(i):
        norms[pl.ds(node0 + i * L, L)] = zero

    @pl.loop(0, nmine // C)
    def _nchunk(t):
        row0 = node0 + t * C
        pltpu.sync_copy(z_hbm.at[pl.ds(row0, C), :], srows0)

        @pl.loop(0, C)
        def _nrow(e):
            pr = [srows0[e, pl.ds(j * L, L)] for j in range(D // L)]
            pr = _tree_sum([v * v for v in pr])
            plsc.addupdate_scatter(
                norms, [jnp.full((L,), row0 + e, jnp.int32)], pr)

    pltpu.sync_copy(norms.at[pl.ds(node0, nmine)],
                    shnorm.at[pl.ds(node0, nmine)])
    plsc.subcore_barrier()
    pltpu.sync_copy(shnorm, norms)

    def issue1(c, p):
        pltpu.async_copy(z_hbm.at[sidx.at[pl.ds(c * C, C)]], srows[p], sg[p])

    def issue2(c, p):
        pltpu.async_copy(z_hbm.at[didx.at[pl.ds(c * C, C)]], srows[p], sg[p],
                         add=True)

    def wait(p):
        pltpu.make_async_copy(
            z_hbm.at[sidx.at[pl.ds(0, C)]], srows[p], sg[p]).wait()

    def compute(c, p):
        sr = srows[p]

        @pl.loop(0, C // L)
        def _blk(b):
            out_base = c * C + b * L
            lo = lanes < (L // 2)
            for e in range(0, L, 2):
                pa = _edge_sq(sr, b * L + e)
                pb = _edge_sq(sr, b * L + e + 1)
                w = (jnp.where(lo, pa, jnp.flip(pb))
                     + jnp.where(lo, jnp.flip(pa), pb))
                idx = jnp.where(lo, out_base + e, out_base + e + 1)
                plsc.addupdate_scatter(outv, [idx], w)
            ns = plsc.load_gather(norms, [sidx[pl.ds(out_base, L)]])
            nd = plsc.load_gather(norms, [didx[pl.ds(out_base, L)]])
            val = 0.5 * (outv[pl.ds(out_base, L)] - ns - nd)
            outv[pl.ds(out_base, L)] = 1.0 / (1.0 + jnp.exp(-val))

    @pl.loop(0, EPW // L)
    def _zero(i):
        outv[pl.ds(i * L, L)] = zero

    issue1(0, 0)
    wait(0)
    issue2(0, 0)
    issue1(1, 1)

    def step(c, p):
        q = (p + 1) % 3
        r = (p + 2) % 3
        wait(q)
        issue2(c + 1, q)
        issue1(c + 2, r)
        wait(p)
        compute(c, p)

    @pl.loop(0, (NCHUNK - 2) // 3)
    def _trip(t):
        c0 = 3 * t
        step(c0, 0)
        step(c0 + 1, 1)
        step(c0 + 2, 2)

    ct = NCHUNK - 2
    pt = ct % 3
    qt = (pt + 1) % 3
    wait(qt)
    issue2(ct + 1, qt)
    wait(pt)
    compute(ct, pt)
    wait(qt)
    compute(ct + 1, qt)

    pltpu.sync_copy(outv, out_hbm.at[pl.ds(base0, EPW)])


@jax.jit
def _run(z, src, dst):
    mesh = plsc.VectorSubcoreMesh(
        core_axis_name="c", subcore_axis_name="s",
        num_cores=NC, num_subcores=NS)
    f = pl.kernel(
        _decoder_body,
        out_type=jax.ShapeDtypeStruct((E,), jnp.float32),
        mesh=mesh,
        scratch_types=[
            pltpu.VMEM((EPW,), jnp.int32),
            pltpu.VMEM((EPW,), jnp.int32),
            pltpu.VMEM((EPW,), jnp.float32),
            pltpu.VMEM((V,), jnp.float32),
            pltpu.VMEM((C, D), jnp.float32),
            pltpu.VMEM((C, D), jnp.float32),
            pltpu.VMEM((C, D), jnp.float32),
            pltpu.VMEM_SHARED((V,), jnp.float32),
            pltpu.SemaphoreType.DMA,
            pltpu.SemaphoreType.DMA,
            pltpu.SemaphoreType.DMA,
        ],
        compiler_params=pltpu.CompilerParams(needs_layout_passes=False),
    )
    return f(z, src, dst)


def kernel(z, edge_index):
    ei = edge_index.astype(jnp.int32)
    return _run(z, ei[0], ei[1])
